# baseline (speedup 1.0000x reference)
import functools

import jax
import jax.numpy as jnp
from jax import lax
from jax.experimental import pallas as pl
from jax.experimental.pallas import tpu as pltpu
from jax.experimental.pallas import tpu_sc as plsc

NUM_USERS = 50000
NUM_ITEMS = 50000
N_NODES = NUM_USERS + NUM_ITEMS
N_EDGES = 1600000
D = 32
H = 16
N_LAYERS = 3
B = 4096

NC = 2
NS = 16
EPT = N_EDGES // NS
BLK = 400
NBLK = EPT // BLK
NB2 = NBLK // 2
N_PAD = 100096
STRIPE = N_PAD // NS
GB = (2 * B) // NS


def _sc_body(emb0, edata, uidx, iidx, zeros_h, e1, e2, e3, out,
             acc, ebuf, rows_a, rows_b,
             semi_a, semi_b, semg_a, semg_b, sems_a, sems_b):
    cid = lax.axis_index("c")
    sid = lax.axis_index("s")
    stripe_base = sid * STRIPE

    def issue_idx(n, semi_s):
        ebase = sid * EPT + n * BLK
        pltpu.async_copy(edata.at[:, pl.ds(ebase, BLK)],
                         ebuf.at[lax.rem(n, 4)], semi_s)

    def wait_idx(n, semi_s):
        ebase = sid * EPT + n * BLK
        pltpu.make_async_copy(edata.at[:, pl.ds(ebase, BLK)],
                              ebuf.at[lax.rem(n, 4)], semi_s).wait()

    def issue_gather(tin, n, rows_s, semg_s):
        pltpu.async_copy(tin.at[cid].at[ebuf.at[lax.rem(n, 4)].at[0]],
                         rows_s, semg_s)

    def wait_gather(tin, n, rows_s, semg_s):
        pltpu.make_async_copy(tin.at[cid].at[ebuf.at[lax.rem(n, 4)].at[0]],
                              rows_s, semg_s).wait()

    def issue_scatter(n, rows_s, sems_s):
        pltpu.async_copy(rows_s, acc.at[ebuf.at[lax.rem(n, 4)].at[1]],
                         sems_s, add=True)

    def wait_scatter(n, rows_s, sems_s):
        pltpu.make_async_copy(rows_s, acc.at[ebuf.at[lax.rem(n, 4)].at[1]],
                              sems_s).wait()

    def multiply(n, rows_s):
        slot = lax.rem(n, 4)

        @plsc.parallel_loop(0, BLK // 16)
        def _mul(g):
            base = g * 16
            wv = plsc.bitcast(ebuf[slot, 2, pl.ds(base, 16)], jnp.float32)
            for i in range(16):
                rows_s[base + i, :] = rows_s[base + i, :] * wv[i]

    for tin, tout in ((emb0, e1), (e1, e2), (e2, e3)):
        pltpu.sync_copy(zeros_h, acc.at[pl.ds(stripe_base, STRIPE)])
        plsc.subcore_barrier()

        issue_idx(0, semi_a)
        issue_idx(1, semi_b)
        wait_idx(0, semi_a)
        issue_gather(tin, 0, rows_a, semg_a)

        def pipe_body(j, _, tin=tin):
            a = 2 * j
            b = a + 1
            wait_gather(tin, a, rows_a, semg_a)

            @pl.when(j > 0)
            def _():
                wait_scatter(a - 1, rows_b, sems_b)

            wait_idx(b, semi_b)
            issue_gather(tin, b, rows_b, semg_b)
            multiply(a, rows_a)
            issue_scatter(a, rows_a, sems_a)

            @pl.when(j < NB2 - 1)
            def _():
                issue_idx(a + 2, semi_a)

            wait_gather(tin, b, rows_b, semg_b)
            wait_scatter(a, rows_a, sems_a)

            @pl.when(j < NB2 - 1)
            def _():
                wait_idx(b + 1, semi_a)
                issue_gather(tin, b + 1, rows_a, semg_a)

            multiply(b, rows_b)
            issue_scatter(b, rows_b, sems_b)

            @pl.when(j < NB2 - 1)
            def _():
                issue_idx(b + 2, semi_b)

            return 0

        lax.fori_loop(0, NB2, pipe_body, 0)
        wait_scatter(NBLK - 1, rows_b, sems_b)
        plsc.subcore_barrier()
        pltpu.sync_copy(acc.at[pl.ds(stripe_base, STRIPE)],
                        tout.at[cid].at[pl.ds(stripe_base, STRIPE)])

    plsc.subcore_barrier()

    FC = 256
    for ch in range(GB // FC):
        obase = sid * GB + ch * FC

        @pl.when(sid < 8)
        def _():
            pltpu.sync_copy(uidx.at[pl.ds(obase, FC)],
                            ebuf.at[0].at[0].at[pl.ds(0, FC)])

        @pl.when(sid >= 8)
        def _():
            pltpu.sync_copy(iidx.at[pl.ds(obase - B, FC)],
                            ebuf.at[0].at[0].at[pl.ds(0, FC)])

            @plsc.parallel_loop(0, FC // 16)
            def _off(r):
                ebuf[0, 0, pl.ds(r * 16, 16)] = (
                    ebuf[0, 0, pl.ds(r * 16, 16)] + jnp.int32(NUM_USERS))

        gidx = ebuf.at[0].at[0].at[pl.ds(0, FC)]
        pltpu.async_copy(emb0.at[cid].at[gidx],
                         rows_b.at[pl.ds(0, FC)], semg_a).wait()
        for tbl in (e1, e2, e3):
            pltpu.async_copy(tbl.at[cid].at[gidx],
                             rows_a.at[pl.ds(0, FC)], semg_a).wait()

            @plsc.parallel_loop(0, FC)
            def _acc(r):
                rows_b[r, :] = rows_b[r, :] + rows_a[r, :]

        quarter = jnp.full((16,), 0.25, jnp.float32)

        @plsc.parallel_loop(0, FC)
        def _avg(r):
            rows_b[r, :] = rows_b[r, :] * quarter

        pltpu.sync_copy(rows_b.at[pl.ds(0, FC)],
                        out.at[cid].at[pl.ds(obase, FC)])


_sc_call = pl.kernel(
    _sc_body,
    out_type=(
        jax.ShapeDtypeStruct((NC, N_PAD, H), jnp.float32),
        jax.ShapeDtypeStruct((NC, N_PAD, H), jnp.float32),
        jax.ShapeDtypeStruct((NC, N_PAD, H), jnp.float32),
        jax.ShapeDtypeStruct((NC, 2 * B, H), jnp.float32),
    ),
    mesh=plsc.VectorSubcoreMesh(core_axis_name="c", subcore_axis_name="s"),
    compiler_params=pltpu.CompilerParams(use_tc_tiling_on_sc=False,
                                         needs_layout_passes=False),
    scratch_types=[
        pltpu.VMEM_SHARED((N_PAD, H), jnp.float32),
        pltpu.VMEM((4, 3, BLK), jnp.int32),
        pltpu.VMEM((BLK, H), jnp.float32),
        pltpu.VMEM((BLK, H), jnp.float32),
        pltpu.SemaphoreType.DMA,
        pltpu.SemaphoreType.DMA,
        pltpu.SemaphoreType.DMA,
        pltpu.SemaphoreType.DMA,
        pltpu.SemaphoreType.DMA,
        pltpu.SemaphoreType.DMA,
    ],
)


def _mlp_body(v_ref, w1_ref, b1_ref, w2_ref, b2_ref, wo_ref, bo_ref, o_ref):
    v = v_ref[...]
    h1 = jnp.maximum(
        jnp.dot(v, w1_ref[...], preferred_element_type=jnp.float32)
        + b1_ref[...], 0.0)
    h2 = jnp.maximum(
        jnp.dot(h1, w2_ref[...], preferred_element_type=jnp.float32)
        + b2_ref[...], 0.0)
    logits = jnp.sum(h2 * wo_ref[...].reshape(1, -1), axis=-1,
                     keepdims=True) + bo_ref[...]
    o_ref[...] = jax.nn.sigmoid(logits)


_mlp_call = pl.pallas_call(
    _mlp_body,
    out_shape=jax.ShapeDtypeStruct((B, 1), jnp.float32),
)


@jax.jit
def kernel(users, items, graph_edge_index, graph_edge_weight,
           user_emb, item_emb, W1, b1, W2, b2, Wo, bo):
    all0 = jnp.concatenate([user_emb, item_emb], axis=0)
    embh = all0.reshape(N_NODES, NC, H).transpose(1, 0, 2)
    edata = jnp.concatenate(
        [graph_edge_index.astype(jnp.int32),
         jax.lax.bitcast_convert_type(graph_edge_weight, jnp.int32)[None]],
        axis=0)

    zeros_h = jnp.zeros((STRIPE, H), jnp.float32)
    _, _, e3_unused, pairs = _sc_call(
        embh, edata,
        users.astype(jnp.int32), items.astype(jnp.int32), zeros_h)
    del e3_unused

    users_emb = jnp.concatenate([pairs[0, :B], pairs[1, :B]], axis=-1)
    items_emb = jnp.concatenate([pairs[0, B:], pairs[1, B:]], axis=-1)
    vector = jnp.concatenate([users_emb, items_emb], axis=-1)

    return _mlp_call(vector, W1, b1.reshape(1, -1), W2, b2.reshape(1, -1),
                     Wo.reshape(-1), bo.reshape(1, 1))


# speedup vs baseline: 1.1667x; 1.1667x over previous
import functools

import jax
import jax.numpy as jnp
from jax import lax
from jax.experimental import pallas as pl
from jax.experimental.pallas import tpu as pltpu
from jax.experimental.pallas import tpu_sc as plsc

NUM_USERS = 50000
NUM_ITEMS = 50000
N_NODES = NUM_USERS + NUM_ITEMS
N_EDGES = 1600000
D = 32
H = 16
N_LAYERS = 3
B = 4096

NC = 2
NS = 16
EPT = N_EDGES // NS
BLK = 400
NBLK = EPT // BLK
NB2 = NBLK // 2
N_PAD = 100096
STRIPE = N_PAD // NS
GB = (2 * B) // NS


def _sc_body(emb0, swdata, ddata, uidx, iidx, zeros_h, e1, e2, e3, out,
             acc, sw_a, sw_b, didx, rows_a, rows_b,
             semi_a, semi_b, semg_a, semg_b, sems_a, sems_b):
    cid = lax.axis_index("c")
    sid = lax.axis_index("s")
    stripe_base = sid * STRIPE

    def issue_idx(n, sw_s, semi_s):
        pltpu.async_copy(swdata.at[sid].at[n], sw_s, semi_s)
        pltpu.async_copy(ddata.at[sid].at[n], didx.at[lax.rem(n, 4)],
                         semi_s)

    def wait_idx(n, sw_s, semi_s):
        pltpu.make_async_copy(swdata.at[sid].at[n], sw_s, semi_s).wait()
        pltpu.make_async_copy(ddata.at[sid].at[n],
                              didx.at[lax.rem(n, 4)], semi_s).wait()

    def issue_gather(tin, sw_s, rows_s, semg_s):
        pltpu.async_copy(tin.at[cid].at[sw_s.at[0]], rows_s, semg_s)

    def wait_gather(tin, sw_s, rows_s, semg_s):
        pltpu.make_async_copy(tin.at[cid].at[sw_s.at[0]], rows_s,
                              semg_s).wait()

    def issue_scatter(n, rows_s, sems_s):
        pltpu.async_copy(rows_s, acc.at[didx.at[lax.rem(n, 4)]], sems_s,
                         add=True)

    def wait_scatter(n, rows_s, sems_s):
        pltpu.make_async_copy(rows_s, acc.at[didx.at[lax.rem(n, 4)]],
                              sems_s).wait()

    def multiply(rows_s, sw_s):
        @plsc.parallel_loop(0, BLK // 16)
        def _mul(g):
            base = g * 16
            wv = plsc.bitcast(sw_s[1, pl.ds(base, 16)], jnp.float32)
            for i in range(16):
                rows_s[base + i, :] = rows_s[base + i, :] * wv[i]

    for tin, tout in ((emb0, e1), (e1, e2), (e2, e3)):
        pltpu.sync_copy(zeros_h, acc.at[pl.ds(stripe_base, STRIPE)])
        plsc.subcore_barrier()

        issue_idx(0, sw_a, semi_a)
        issue_idx(1, sw_b, semi_b)
        wait_idx(0, sw_a, semi_a)
        issue_gather(tin, sw_a, rows_a, semg_a)

        def pipe_body(j, _, tin=tin):
            a = 2 * j
            b = a + 1
            wait_gather(tin, sw_a, rows_a, semg_a)

            @pl.when(j > 0)
            def _():
                wait_scatter(a - 1, rows_b, sems_b)

            wait_idx(b, sw_b, semi_b)
            issue_gather(tin, sw_b, rows_b, semg_b)
            multiply(rows_a, sw_a)
            issue_scatter(a, rows_a, sems_a)

            @pl.when(j < NB2 - 1)
            def _():
                issue_idx(a + 2, sw_a, semi_a)

            wait_gather(tin, sw_b, rows_b, semg_b)
            wait_scatter(a, rows_a, sems_a)

            @pl.when(j < NB2 - 1)
            def _():
                wait_idx(b + 1, sw_a, semi_a)
                issue_gather(tin, sw_a, rows_a, semg_a)

            multiply(rows_b, sw_b)
            issue_scatter(b, rows_b, sems_b)

            @pl.when(j < NB2 - 1)
            def _():
                issue_idx(b + 2, sw_b, semi_b)

            return 0

        lax.fori_loop(0, NB2, pipe_body, 0)
        wait_scatter(NBLK - 1, rows_b, sems_b)
        plsc.subcore_barrier()
        pltpu.sync_copy(acc.at[pl.ds(stripe_base, STRIPE)],
                        tout.at[cid].at[pl.ds(stripe_base, STRIPE)])

    plsc.subcore_barrier()

    FC = 256
    for ch in range(GB // FC):
        obase = sid * GB + ch * FC

        @pl.when(sid < 8)
        def _():
            pltpu.sync_copy(uidx.at[pl.ds(obase, FC)],
                            sw_a.at[0].at[pl.ds(0, FC)])

        @pl.when(sid >= 8)
        def _():
            pltpu.sync_copy(iidx.at[pl.ds(obase - B, FC)],
                            sw_a.at[0].at[pl.ds(0, FC)])

            @plsc.parallel_loop(0, FC // 16)
            def _off(r):
                sw_a[0, pl.ds(r * 16, 16)] = (
                    sw_a[0, pl.ds(r * 16, 16)] + jnp.int32(NUM_USERS))

        gidx = sw_a.at[0].at[pl.ds(0, FC)]
        pltpu.async_copy(emb0.at[cid].at[gidx],
                         rows_b.at[pl.ds(0, FC)], semg_a).wait()
        for tbl in (e1, e2, e3):
            pltpu.async_copy(tbl.at[cid].at[gidx],
                             rows_a.at[pl.ds(0, FC)], semg_a).wait()

            @plsc.parallel_loop(0, FC)
            def _acc(r):
                rows_b[r, :] = rows_b[r, :] + rows_a[r, :]

        quarter = jnp.full((16,), 0.25, jnp.float32)

        @plsc.parallel_loop(0, FC)
        def _avg(r):
            rows_b[r, :] = rows_b[r, :] * quarter

        pltpu.sync_copy(rows_b.at[pl.ds(0, FC)],
                        out.at[cid].at[pl.ds(obase, FC)])


_sc_call = pl.kernel(
    _sc_body,
    out_type=(
        jax.ShapeDtypeStruct((NC, N_PAD, H), jnp.float32),
        jax.ShapeDtypeStruct((NC, N_PAD, H), jnp.float32),
        jax.ShapeDtypeStruct((NC, N_PAD, H), jnp.float32),
        jax.ShapeDtypeStruct((NC, 2 * B, H), jnp.float32),
    ),
    mesh=plsc.VectorSubcoreMesh(core_axis_name="c", subcore_axis_name="s"),
    compiler_params=pltpu.CompilerParams(use_tc_tiling_on_sc=False,
                                         needs_layout_passes=False),
    scratch_types=[
        pltpu.VMEM_SHARED((N_PAD, H), jnp.float32),
        pltpu.VMEM((2, BLK), jnp.int32),
        pltpu.VMEM((2, BLK), jnp.int32),
        pltpu.VMEM((4, BLK), jnp.int32),
        pltpu.VMEM((BLK, H), jnp.float32),
        pltpu.VMEM((BLK, H), jnp.float32),
        pltpu.SemaphoreType.DMA,
        pltpu.SemaphoreType.DMA,
        pltpu.SemaphoreType.DMA,
        pltpu.SemaphoreType.DMA,
        pltpu.SemaphoreType.DMA,
        pltpu.SemaphoreType.DMA,
    ],
)


def _mlp_body(v_ref, w1_ref, b1_ref, w2_ref, b2_ref, wo_ref, bo_ref, o_ref):
    v = v_ref[...]
    h1 = jnp.maximum(
        jnp.dot(v, w1_ref[...], preferred_element_type=jnp.float32)
        + b1_ref[...], 0.0)
    h2 = jnp.maximum(
        jnp.dot(h1, w2_ref[...], preferred_element_type=jnp.float32)
        + b2_ref[...], 0.0)
    logits = jnp.sum(h2 * wo_ref[...].reshape(1, -1), axis=-1,
                     keepdims=True) + bo_ref[...]
    o_ref[...] = jax.nn.sigmoid(logits)


_mlp_call = pl.pallas_call(
    _mlp_body,
    out_shape=jax.ShapeDtypeStruct((B, 1), jnp.float32),
)


@jax.jit
def kernel(users, items, graph_edge_index, graph_edge_weight,
           user_emb, item_emb, W1, b1, W2, b2, Wo, bo):
    all0 = jnp.concatenate([user_emb, item_emb], axis=0)
    embh = all0.reshape(N_NODES, NC, H).transpose(1, 0, 2)
    src = graph_edge_index[0].astype(jnp.int32)
    dst = graph_edge_index[1].astype(jnp.int32)
    wbits = jax.lax.bitcast_convert_type(graph_edge_weight, jnp.int32)
    swdata = jnp.stack([src.reshape(NS, NBLK, BLK),
                        wbits.reshape(NS, NBLK, BLK)], axis=2)
    ddata = dst.reshape(NS, NBLK, BLK)

    zeros_h = jnp.zeros((STRIPE, H), jnp.float32)
    _, _, e3_unused, pairs = _sc_call(
        embh, swdata, ddata,
        users.astype(jnp.int32), items.astype(jnp.int32), zeros_h)
    del e3_unused

    users_emb = jnp.concatenate([pairs[0, :B], pairs[1, :B]], axis=-1)
    items_emb = jnp.concatenate([pairs[0, B:], pairs[1, B:]], axis=-1)
    vector = jnp.concatenate([users_emb, items_emb], axis=-1)

    return _mlp_call(vector, W1, b1.reshape(1, -1), W2, b2.reshape(1, -1),
                     Wo.reshape(-1), bo.reshape(1, 1))


# speedup vs baseline: 1.3113x; 1.1240x over previous
import functools

import jax
import jax.numpy as jnp
from jax import lax
from jax.experimental import pallas as pl
from jax.experimental.pallas import tpu as pltpu
from jax.experimental.pallas import tpu_sc as plsc

NUM_USERS = 50000
NUM_ITEMS = 50000
N_NODES = NUM_USERS + NUM_ITEMS
N_EDGES = 1600000
D = 32
H = 16
N_LAYERS = 3
B = 4096

NC = 2
NS = 16
EPT = N_EDGES // NS
BLK = 400
NBLK = EPT // BLK
NB2 = NBLK // 2
N_PAD = 100096
STRIPE = N_PAD // NS
GB = (2 * B) // NS


def _sc_body(emb0, src, dst, w, uidx, iidx, zeros_h, e1, e2, e3, out,
             acc, sidx_a, sidx_b, didx, w_a, w_b, rows_a, rows_b,
             semi_a, semi_b, semg_a, semg_b, sems_a, sems_b):
    cid = lax.axis_index("c")
    sid = lax.axis_index("s")
    stripe_base = sid * STRIPE

    def issue_idx(n, sidx_s, w_s, semi_s):
        ebase = sid * EPT + n * BLK
        pltpu.async_copy(src.at[pl.ds(ebase, BLK)], sidx_s, semi_s)
        pltpu.async_copy(dst.at[pl.ds(ebase, BLK)], didx.at[lax.rem(n, 4)],
                         semi_s)
        pltpu.async_copy(w.at[pl.ds(ebase, BLK)], w_s.at[pl.ds(0, BLK)],
                         semi_s)

    def wait_idx(n, sidx_s, w_s, semi_s):
        ebase = sid * EPT + n * BLK
        pltpu.make_async_copy(src.at[pl.ds(ebase, BLK)], sidx_s,
                              semi_s).wait()
        pltpu.make_async_copy(dst.at[pl.ds(ebase, BLK)],
                              didx.at[lax.rem(n, 4)], semi_s).wait()
        pltpu.make_async_copy(w.at[pl.ds(ebase, BLK)],
                              w_s.at[pl.ds(0, BLK)], semi_s).wait()

    HB = BLK // 2

    def issue_gather(tin, sidx_s, rows_s, semg_s):
        pltpu.async_copy(tin.at[cid].at[sidx_s.at[pl.ds(0, HB)]],
                         rows_s.at[pl.ds(0, HB)], semg_s)
        pltpu.async_copy(tin.at[cid].at[sidx_s.at[pl.ds(HB, HB)]],
                         rows_s.at[pl.ds(HB, HB)], semg_s)

    def wait_gather(tin, sidx_s, rows_s, semg_s):
        pltpu.make_async_copy(tin.at[cid].at[sidx_s.at[pl.ds(0, HB)]],
                              rows_s.at[pl.ds(0, HB)], semg_s).wait()
        pltpu.make_async_copy(tin.at[cid].at[sidx_s.at[pl.ds(HB, HB)]],
                              rows_s.at[pl.ds(HB, HB)], semg_s).wait()

    def issue_scatter(n, rows_s, sems_s):
        pltpu.async_copy(rows_s, acc.at[didx.at[lax.rem(n, 4)]], sems_s,
                         add=True)

    def wait_scatter(n, rows_s, sems_s):
        pltpu.make_async_copy(rows_s, acc.at[didx.at[lax.rem(n, 4)]],
                              sems_s).wait()

    def multiply(rows_s, w_s):
        @plsc.parallel_loop(0, BLK // 16)
        def _mul(g):
            base = g * 16
            wv = w_s[pl.ds(base, 16)]
            for i in range(16):
                rows_s[base + i, :] = rows_s[base + i, :] * wv[i]

    for tin, tout in ((emb0, e1), (e1, e2), (e2, e3)):
        pltpu.sync_copy(zeros_h, acc.at[pl.ds(stripe_base, STRIPE)])
        plsc.subcore_barrier()

        issue_idx(0, sidx_a, w_a, semi_a)
        issue_idx(1, sidx_b, w_b, semi_b)
        wait_idx(0, sidx_a, w_a, semi_a)
        issue_gather(tin, sidx_a, rows_a, semg_a)

        def pipe_body(j, _, tin=tin):
            a = 2 * j
            b = a + 1
            wait_gather(tin, sidx_a, rows_a, semg_a)

            @pl.when(j > 0)
            def _():
                wait_scatter(a - 1, rows_b, sems_b)

            wait_idx(b, sidx_b, w_b, semi_b)
            issue_gather(tin, sidx_b, rows_b, semg_b)
            multiply(rows_a, w_a)
            issue_scatter(a, rows_a, sems_a)

            @pl.when(j < NB2 - 1)
            def _():
                issue_idx(a + 2, sidx_a, w_a, semi_a)

            wait_gather(tin, sidx_b, rows_b, semg_b)
            wait_scatter(a, rows_a, sems_a)

            @pl.when(j < NB2 - 1)
            def _():
                wait_idx(b + 1, sidx_a, w_a, semi_a)
                issue_gather(tin, sidx_a, rows_a, semg_a)

            multiply(rows_b, w_b)
            issue_scatter(b, rows_b, sems_b)

            @pl.when(j < NB2 - 1)
            def _():
                issue_idx(b + 2, sidx_b, w_b, semi_b)

            return 0

        lax.fori_loop(0, NB2, pipe_body, 0)
        wait_scatter(NBLK - 1, rows_b, sems_b)
        plsc.subcore_barrier()
        pltpu.sync_copy(acc.at[pl.ds(stripe_base, STRIPE)],
                        tout.at[cid].at[pl.ds(stripe_base, STRIPE)])

    plsc.subcore_barrier()

    FC = 256
    for ch in range(GB // FC):
        obase = sid * GB + ch * FC

        @pl.when(sid < 8)
        def _():
            pltpu.sync_copy(uidx.at[pl.ds(obase, FC)],
                            sidx_a.at[pl.ds(0, FC)])

        @pl.when(sid >= 8)
        def _():
            pltpu.sync_copy(iidx.at[pl.ds(obase - B, FC)],
                            sidx_a.at[pl.ds(0, FC)])

            @plsc.parallel_loop(0, FC // 16)
            def _off(r):
                sidx_a[pl.ds(r * 16, 16)] = (
                    sidx_a[pl.ds(r * 16, 16)] + jnp.int32(NUM_USERS))

        gidx = sidx_a.at[pl.ds(0, FC)]
        pltpu.async_copy(emb0.at[cid].at[gidx],
                         rows_b.at[pl.ds(0, FC)], semg_a).wait()
        for tbl in (e1, e2, e3):
            pltpu.async_copy(tbl.at[cid].at[gidx],
                             rows_a.at[pl.ds(0, FC)], semg_a).wait()

            @plsc.parallel_loop(0, FC)
            def _acc(r):
                rows_b[r, :] = rows_b[r, :] + rows_a[r, :]

        quarter = jnp.full((16,), 0.25, jnp.float32)

        @plsc.parallel_loop(0, FC)
        def _avg(r):
            rows_b[r, :] = rows_b[r, :] * quarter

        pltpu.sync_copy(rows_b.at[pl.ds(0, FC)],
                        out.at[cid].at[pl.ds(obase, FC)])


_sc_call = pl.kernel(
    _sc_body,
    out_type=(
        jax.ShapeDtypeStruct((NC, N_PAD, H), jnp.float32),
        jax.ShapeDtypeStruct((NC, N_PAD, H), jnp.float32),
        jax.ShapeDtypeStruct((NC, N_PAD, H), jnp.float32),
        jax.ShapeDtypeStruct((NC, 2 * B, H), jnp.float32),
    ),
    mesh=plsc.VectorSubcoreMesh(core_axis_name="c", subcore_axis_name="s"),
    compiler_params=pltpu.CompilerParams(use_tc_tiling_on_sc=False,
                                         needs_layout_passes=False),
    scratch_types=[
        pltpu.VMEM_SHARED((N_PAD, H), jnp.float32),
        pltpu.VMEM((BLK,), jnp.int32),
        pltpu.VMEM((BLK,), jnp.int32),
        pltpu.VMEM((4, BLK), jnp.int32),
        pltpu.VMEM((BLK + 16,), jnp.float32),
        pltpu.VMEM((BLK + 16,), jnp.float32),
        pltpu.VMEM((BLK, H), jnp.float32),
        pltpu.VMEM((BLK, H), jnp.float32),
        pltpu.SemaphoreType.DMA,
        pltpu.SemaphoreType.DMA,
        pltpu.SemaphoreType.DMA,
        pltpu.SemaphoreType.DMA,
        pltpu.SemaphoreType.DMA,
        pltpu.SemaphoreType.DMA,
    ],
)


def _mlp_body(v_ref, w1_ref, b1_ref, w2_ref, b2_ref, wo_ref, bo_ref, o_ref):
    v = v_ref[...]
    h1 = jnp.maximum(
        jnp.dot(v, w1_ref[...], preferred_element_type=jnp.float32)
        + b1_ref[...], 0.0)
    h2 = jnp.maximum(
        jnp.dot(h1, w2_ref[...], preferred_element_type=jnp.float32)
        + b2_ref[...], 0.0)
    logits = jnp.sum(h2 * wo_ref[...].reshape(1, -1), axis=-1,
                     keepdims=True) + bo_ref[...]
    o_ref[...] = jax.nn.sigmoid(logits)


_mlp_call = pl.pallas_call(
    _mlp_body,
    out_shape=jax.ShapeDtypeStruct((B, 1), jnp.float32),
)


@jax.jit
def kernel(users, items, graph_edge_index, graph_edge_weight,
           user_emb, item_emb, W1, b1, W2, b2, Wo, bo):
    all0 = jnp.concatenate([user_emb, item_emb], axis=0)
    embh = all0.reshape(N_NODES, NC, H).transpose(1, 0, 2)
    src = graph_edge_index[0]
    dst = graph_edge_index[1]

    zeros_h = jnp.zeros((STRIPE, H), jnp.float32)
    _, _, e3_unused, pairs = _sc_call(
        embh, src, dst, graph_edge_weight,
        users.astype(jnp.int32), items.astype(jnp.int32), zeros_h)
    del e3_unused

    users_emb = jnp.concatenate([pairs[0, :B], pairs[1, :B]], axis=-1)
    items_emb = jnp.concatenate([pairs[0, B:], pairs[1, B:]], axis=-1)
    vector = jnp.concatenate([users_emb, items_emb], axis=-1)

    return _mlp_call(vector, W1, b1.reshape(1, -1), W2, b2.reshape(1, -1),
                     Wo.reshape(-1), bo.reshape(1, 1))


# speedup vs baseline: 1.3119x; 1.0004x over previous
import jax
import jax.numpy as jnp
from jax import lax
from jax.experimental import pallas as pl
from jax.experimental.pallas import tpu as pltpu
from jax.experimental.pallas import tpu_sc as plsc

NUM_USERS = 50000
NUM_ITEMS = 50000
N_NODES = NUM_USERS + NUM_ITEMS
N_EDGES = 1600000
D = 32
H = 16
N_LAYERS = 3
B = 4096

NC = 2
NS = 16
EPT = N_EDGES // NS
BLK = 400
NBLK = EPT // BLK
NB2 = NBLK // 2
N_PAD = 100096
STRIPE = N_PAD // NS
GB = (2 * B) // NS


def _sc_body(emb0, src, dst, w, uidx, iidx, zeros_h, e1, e2, e3, out,
             acc, sidx_a, sidx_b, didx, w_a, w_b, rows_a, rows_b,
             semi_a, semi_b, semg_a, semg_b, sems_a, sems_b):
    cid = lax.axis_index("c")
    sid = lax.axis_index("s")
    stripe_base = sid * STRIPE

    def issue_idx(n, sidx_s, w_s, semi_s):
        ebase = sid * EPT + n * BLK
        pltpu.async_copy(src.at[pl.ds(ebase, BLK)], sidx_s, semi_s)
        pltpu.async_copy(dst.at[pl.ds(ebase, BLK)], didx.at[lax.rem(n, 4)],
                         semi_s)
        pltpu.async_copy(w.at[pl.ds(ebase, BLK)], w_s.at[pl.ds(0, BLK)],
                         semi_s)

    def wait_idx(n, sidx_s, w_s, semi_s):
        ebase = sid * EPT + n * BLK
        pltpu.make_async_copy(src.at[pl.ds(ebase, BLK)], sidx_s,
                              semi_s).wait()
        pltpu.make_async_copy(dst.at[pl.ds(ebase, BLK)],
                              didx.at[lax.rem(n, 4)], semi_s).wait()
        pltpu.make_async_copy(w.at[pl.ds(ebase, BLK)],
                              w_s.at[pl.ds(0, BLK)], semi_s).wait()

    HB = BLK // 2

    def issue_gather(tin, sidx_s, rows_s, semg_s):
        pltpu.async_copy(tin.at[cid].at[sidx_s.at[pl.ds(0, HB)]],
                         rows_s.at[pl.ds(0, HB)], semg_s)
        pltpu.async_copy(tin.at[cid].at[sidx_s.at[pl.ds(HB, HB)]],
                         rows_s.at[pl.ds(HB, HB)], semg_s)

    def wait_gather(tin, sidx_s, rows_s, semg_s):
        pltpu.make_async_copy(tin.at[cid].at[sidx_s.at[pl.ds(0, HB)]],
                              rows_s.at[pl.ds(0, HB)], semg_s).wait()
        pltpu.make_async_copy(tin.at[cid].at[sidx_s.at[pl.ds(HB, HB)]],
                              rows_s.at[pl.ds(HB, HB)], semg_s).wait()

    def issue_scatter(n, rows_s, sems_s):
        pltpu.async_copy(rows_s, acc.at[didx.at[lax.rem(n, 4)]], sems_s,
                         add=True)

    def wait_scatter(n, rows_s, sems_s):
        pltpu.make_async_copy(rows_s, acc.at[didx.at[lax.rem(n, 4)]],
                              sems_s).wait()

    def multiply(rows_s, w_s):
        @plsc.parallel_loop(0, BLK // 16)
        def _mul(g):
            base = g * 16
            wv = w_s[pl.ds(base, 16)]
            for i in range(16):
                rows_s[base + i, :] = rows_s[base + i, :] * wv[i]

    for tin, tout in ((emb0, e1), (e1, e2), (e2, e3)):
        pltpu.sync_copy(zeros_h, acc.at[pl.ds(stripe_base, STRIPE)])
        plsc.subcore_barrier()

        issue_idx(0, sidx_a, w_a, semi_a)
        issue_idx(1, sidx_b, w_b, semi_b)
        wait_idx(0, sidx_a, w_a, semi_a)
        issue_gather(tin, sidx_a, rows_a, semg_a)

        def pipe_body(j, _, tin=tin):
            a = 2 * j
            b = a + 1
            wait_gather(tin, sidx_a, rows_a, semg_a)

            @pl.when(j > 0)
            def _():
                wait_scatter(a - 1, rows_b, sems_b)

            wait_idx(b, sidx_b, w_b, semi_b)
            issue_gather(tin, sidx_b, rows_b, semg_b)
            multiply(rows_a, w_a)
            issue_scatter(a, rows_a, sems_a)

            @pl.when(j < NB2 - 1)
            def _():
                issue_idx(a + 2, sidx_a, w_a, semi_a)

            wait_gather(tin, sidx_b, rows_b, semg_b)
            wait_scatter(a, rows_a, sems_a)

            @pl.when(j < NB2 - 1)
            def _():
                wait_idx(b + 1, sidx_a, w_a, semi_a)
                issue_gather(tin, sidx_a, rows_a, semg_a)

            multiply(rows_b, w_b)
            issue_scatter(b, rows_b, sems_b)

            @pl.when(j < NB2 - 1)
            def _():
                issue_idx(b + 2, sidx_b, w_b, semi_b)

            return 0

        lax.fori_loop(0, NB2, pipe_body, 0)
        wait_scatter(NBLK - 1, rows_b, sems_b)
        plsc.subcore_barrier()
        pltpu.sync_copy(acc.at[pl.ds(stripe_base, STRIPE)],
                        tout.at[cid].at[pl.ds(stripe_base, STRIPE)])

    plsc.subcore_barrier()

    FC = 256
    for ch in range(GB // FC):
        obase = sid * GB + ch * FC

        @pl.when(sid < 8)
        def _():
            pltpu.sync_copy(uidx.at[pl.ds(obase, FC)],
                            sidx_a.at[pl.ds(0, FC)])

        @pl.when(sid >= 8)
        def _():
            pltpu.sync_copy(iidx.at[pl.ds(obase - B, FC)],
                            sidx_a.at[pl.ds(0, FC)])

            @plsc.parallel_loop(0, FC // 16)
            def _off(r):
                sidx_a[pl.ds(r * 16, 16)] = (
                    sidx_a[pl.ds(r * 16, 16)] + jnp.int32(NUM_USERS))

        gidx = sidx_a.at[pl.ds(0, FC)]
        pltpu.async_copy(emb0.at[cid].at[gidx],
                         rows_b.at[pl.ds(0, FC)], semg_a).wait()
        for tbl in (e1, e2, e3):
            pltpu.async_copy(tbl.at[cid].at[gidx],
                             rows_a.at[pl.ds(0, FC)], semg_a).wait()

            @plsc.parallel_loop(0, FC)
            def _acc(r):
                rows_b[r, :] = rows_b[r, :] + rows_a[r, :]

        quarter = jnp.full((16,), 0.25, jnp.float32)

        @plsc.parallel_loop(0, FC)
        def _avg(r):
            rows_b[r, :] = rows_b[r, :] * quarter

        pltpu.sync_copy(rows_b.at[pl.ds(0, FC)],
                        out.at[cid].at[pl.ds(obase, FC)])


_sc_call = pl.kernel(
    _sc_body,
    out_type=(
        jax.ShapeDtypeStruct((NC, N_PAD, H), jnp.float32),
        jax.ShapeDtypeStruct((NC, N_PAD, H), jnp.float32),
        jax.ShapeDtypeStruct((NC, N_PAD, H), jnp.float32),
        jax.ShapeDtypeStruct((NC, 2 * B, H), jnp.float32),
    ),
    mesh=plsc.VectorSubcoreMesh(core_axis_name="c", subcore_axis_name="s"),
    compiler_params=pltpu.CompilerParams(use_tc_tiling_on_sc=False,
                                         needs_layout_passes=False),
    scratch_types=[
        pltpu.VMEM_SHARED((N_PAD, H), jnp.float32),
        pltpu.VMEM((BLK,), jnp.int32),
        pltpu.VMEM((BLK,), jnp.int32),
        pltpu.VMEM((4, BLK), jnp.int32),
        pltpu.VMEM((BLK + 16,), jnp.float32),
        pltpu.VMEM((BLK + 16,), jnp.float32),
        pltpu.VMEM((BLK, H), jnp.float32),
        pltpu.VMEM((BLK, H), jnp.float32),
        pltpu.SemaphoreType.DMA,
        pltpu.SemaphoreType.DMA,
        pltpu.SemaphoreType.DMA,
        pltpu.SemaphoreType.DMA,
        pltpu.SemaphoreType.DMA,
        pltpu.SemaphoreType.DMA,
    ],
)


def _mlp_body(v_ref, w1_ref, b1_ref, w2_ref, b2_ref, wo_ref, bo_ref, o_ref):
    v = v_ref[...]
    h1 = jnp.maximum(
        jnp.dot(v, w1_ref[...], preferred_element_type=jnp.float32)
        + b1_ref[...], 0.0)
    h2 = jnp.maximum(
        jnp.dot(h1, w2_ref[...], preferred_element_type=jnp.float32)
        + b2_ref[...], 0.0)
    logits = jnp.sum(h2 * wo_ref[...].reshape(1, -1), axis=-1,
                     keepdims=True) + bo_ref[...]
    o_ref[...] = jax.nn.sigmoid(logits)


_mlp_call = pl.pallas_call(
    _mlp_body,
    out_shape=jax.ShapeDtypeStruct((B, 1), jnp.float32),
)


@jax.jit
def kernel(users, items, graph_edge_index, graph_edge_weight,
           user_emb, item_emb, W1, b1, W2, b2, Wo, bo):
    all0 = jnp.concatenate([user_emb, item_emb], axis=0)
    embh = all0.reshape(N_NODES, NC, H).transpose(1, 0, 2)
    src = graph_edge_index[0]
    dst = graph_edge_index[1]

    zeros_h = jnp.zeros((STRIPE, H), jnp.float32)
    _, _, e3_unused, pairs = _sc_call(
        embh, src, dst, graph_edge_weight,
        users.astype(jnp.int32), items.astype(jnp.int32), zeros_h)
    del e3_unused

    users_emb = jnp.concatenate([pairs[0, :B], pairs[1, :B]], axis=-1)
    items_emb = jnp.concatenate([pairs[0, B:], pairs[1, B:]], axis=-1)
    vector = jnp.concatenate([users_emb, items_emb], axis=-1)

    return _mlp_call(vector, W1, b1.reshape(1, -1), W2, b2.reshape(1, -1),
                     Wo.reshape(-1), bo.reshape(1, 1))
